# baseline (speedup 1.0000x reference)
import functools

import jax
import jax.numpy as jnp
from jax import lax
from jax.experimental import pallas as pl
from jax.experimental.pallas import tpu as pltpu
from jax.experimental.pallas import tpu_sc as plsc

_D = 128
_BATCH = 4096
_SEQ = 50
_NC = 2
_NS = 16
_NW = _NC * _NS
_RPW = _BATCH // _NW
_NBUF = 7
_LAG = 2

_mesh = plsc.VectorSubcoreMesh(core_axis_name="c", subcore_axis_name="s")


@functools.partial(
    pl.kernel,
    mesh=_mesh,
    out_type=jax.ShapeDtypeStruct((_SEQ, _BATCH, _D), jnp.float32),
    scratch_types=[
        pltpu.VMEM((_SEQ, _RPW), jnp.int32),
        pltpu.VMEM((_NBUF, _RPW, _D), jnp.float32),
        pltpu.SemaphoreType.DMA,
        pltpu.SemaphoreType.DMA,
    ],
)
def _embed_lookup(idx_hbm, table_hbm, out_hbm, idx_v, rows_v, gsem, ssem):
    wid = lax.axis_index("s") * _NC + lax.axis_index("c")
    base = wid * _RPW
    pltpu.sync_copy(idx_hbm.at[:, wid], idx_v)

    def gather(j, b):
        return pltpu.make_async_copy(
            table_hbm.at[idx_v.at[j]], rows_v.at[b], gsem)

    def scatter(j, b):
        return pltpu.make_async_copy(
            rows_v.at[b], out_hbm.at[j, pl.ds(base, _RPW)], ssem)

    def gather_d(j):
        return pltpu.make_async_copy(
            table_hbm.at[idx_v.at[j]], rows_v.at[lax.rem(j, _NBUF)], gsem)

    def scatter_d(j):
        return pltpu.make_async_copy(
            rows_v.at[lax.rem(j, _NBUF)], out_hbm.at[j, pl.ds(base, _RPW)],
            ssem)

    for b in range(_NBUF):
        gather(b, b).start()

    for j in range(_LAG):
        gather(j, j).wait()
        scatter(j, j).start()

    @pl.loop(_LAG, _SEQ - _NBUF + _LAG)
    def _step(j):
        gather_d(j).wait()
        scatter_d(j).start()
        scatter_d(j - _LAG).wait()
        gather_d(j - _LAG + _NBUF).start()

    for j in range(_SEQ - _NBUF + _LAG, _SEQ):
        b = j % _NBUF
        gather(j, b).wait()
        scatter(j, b).start()
        scatter(j - _LAG, (j - _LAG) % _NBUF).wait()
    for j in range(_SEQ - _LAG, _SEQ):
        scatter(j, j % _NBUF).wait()


def kernel(x, table):
    idx = jnp.swapaxes(x, 0, 1).reshape(_SEQ, _NW, _RPW)
    raw = _embed_lookup(idx, table)
    return jnp.swapaxes(raw, 0, 1)
